# baseline (speedup 1.0000x reference)
import functools

import jax
import jax.numpy as jnp
from jax import lax
from jax.experimental import pallas as pl
from jax.experimental.pallas import tpu as pltpu
from jax.experimental.pallas import tpu_sc as plsc

NUM_FIELDS = 26
FIELD_DIM = 100000
EMBED_DIM = 16
DEEP_IN = NUM_FIELDS * EMBED_DIM
DEEP_OUT = 400
BATCH = 16384
TOTAL_IDX = BATCH * NUM_FIELDS

NC = 2
NS = 16
NW = NC * NS
PER_W = TOTAL_IDX // NW
FPAD = 32
EMB_IDX = BATCH * FPAD
PER_WE = EMB_IDX // NW
ECHUNK = 512
LCHUNK = 416
NCHUNK = PER_WE // ECHUNK

BB = 1024
NB = BATCH // BB


TR_CB = 131072
TR_NB = (2600000 + TR_CB - 1) // TR_CB
TAB_ROWS_PAD = TR_NB * TR_CB


def _tr_body(in_ref, out_ref):
    x = in_ref[...]
    w8 = TR_CB // 8
    m = jnp.concatenate(
        [x[:, w8 * a:w8 * (a + 1)] for a in range(8)], axis=0)
    out_ref[...] = jnp.transpose(m, (1, 0))


def _tc_transpose(embT):
    return pl.pallas_call(
        _tr_body,
        grid=(TR_NB,),
        in_specs=[pl.BlockSpec((EMBED_DIM, TR_CB), lambda i: (0, i))],
        out_specs=pl.BlockSpec((TR_CB // 8, 128), lambda i: (i, 0)),
        out_shape=jax.ShapeDtypeStruct((TR_NB * (TR_CB // 8), 128), jnp.float32),
    )(embT)


def _permute_idx(r):
    return (r & ~(TR_CB - 1)) | ((r & (TR_CB // 8 - 1)) << 3) | ((r >> 14) & 7)


def _sc_gather(xi_flat, xip_flat, emb_table, lin16):
    mesh = plsc.VectorSubcoreMesh(core_axis_name="c", subcore_axis_name="s")

    @functools.partial(
        pl.kernel,
        mesh=mesh,
        compiler_params=pltpu.CompilerParams(use_tc_tiling_on_sc=False,
                                             needs_layout_passes=False),
        out_type=(
            jax.ShapeDtypeStruct((EMB_IDX, EMBED_DIM), jnp.float32),
            jax.ShapeDtypeStruct((TOTAL_IDX,), jnp.float32),
        ),
        scratch_types=[
            pltpu.VMEM((2, LCHUNK), jnp.int32),
            pltpu.VMEM((2, ECHUNK), jnp.int32),
            pltpu.VMEM((2, LCHUNK), jnp.int32),
            pltpu.VMEM((2, ECHUNK, EMBED_DIM), jnp.float32),
            pltpu.VMEM((2, LCHUNK, EMBED_DIM), jnp.float32),
            pltpu.VMEM((2, LCHUNK), jnp.float32),
            pltpu.SemaphoreType.DMA((2,)),
            pltpu.SemaphoreType.DMA((2,)),
            pltpu.SemaphoreType.DMA((2,)),
            pltpu.SemaphoreType.DMA((2,)),
            pltpu.SemaphoreType.DMA((2,)),
        ],
    )
    def k(xi_hbm, xip_hbm, emb_hbm, lin_hbm, emb_out, lin_out, idx_v, idxp_v,
          hi_v, rows_v, linrows_v, linval_v, sin, sge, sgl, swe, swl):
        wid = lax.axis_index("s") * NC + lax.axis_index("c")
        base_wl = wid * PER_W
        base_we = wid * PER_WE
        lanes = lax.iota(jnp.int32, 16)

        def fire_in(ci, b):
            pltpu.async_copy(xi_hbm.at[pl.ds(base_wl + ci * LCHUNK, LCHUNK)],
                             idx_v.at[b], sin.at[b])
            pltpu.async_copy(xip_hbm.at[pl.ds(base_we + ci * ECHUNK, ECHUNK)],
                             idxp_v.at[b], sin.at[b])

        fire_in(0, 0)

        def body(j, carry):
            for b in range(2):
                ci = 2 * j + b
                basel = base_wl + ci * LCHUNK
                basee = base_we + ci * ECHUNK
                pltpu.make_async_copy(xi_hbm.at[pl.ds(basel, LCHUNK)],
                                      idx_v.at[b], sin.at[b]).wait()
                pltpu.make_async_copy(xip_hbm.at[pl.ds(basee, ECHUNK)],
                                      idxp_v.at[b], sin.at[b]).wait()
                @pl.when(j > 0)
                def _():
                    pltpu.make_async_copy(rows_v.at[b],
                                          emb_out.at[pl.ds(0, ECHUNK)],
                                          swe.at[b]).wait()
                    pltpu.make_async_copy(linval_v.at[b],
                                          lin_out.at[pl.ds(0, LCHUNK)],
                                          swl.at[b]).wait()
                ge = pltpu.async_copy(emb_hbm.at[idxp_v.at[b]], rows_v.at[b],
                                      sge.at[b])

                def hi_body(g, c):
                    sl = pl.ds(g * 16, 16)
                    hi_v[b, sl] = lax.shift_right_logical(idx_v[b, sl], 4)
                    return c

                lax.fori_loop(0, LCHUNK // 16, hi_body, 0)
                gl = pltpu.async_copy(lin_hbm.at[hi_v.at[b]],
                                      linrows_v.at[b], sgl.at[b])
                @pl.when(ci + 1 < NCHUNK)
                def _():
                    fire_in(ci + 1, 1 - b)
                ge.wait()
                pltpu.async_copy(rows_v.at[b],
                                 emb_out.at[pl.ds(basee, ECHUNK)], swe.at[b])
                gl.wait()

                def sel_body(g, c):
                    sl = pl.ds(g * 16, 16)
                    lane = lax.bitwise_and(idx_v[b, sl], 15)
                    row = lanes + g * 16
                    linval_v[b, sl] = plsc.load_gather(linrows_v.at[b],
                                                       [row, lane])
                    return c

                lax.fori_loop(0, LCHUNK // 16, sel_body, 0)
                pltpu.async_copy(linval_v.at[b],
                                 lin_out.at[pl.ds(basel, LCHUNK)], swl.at[b])
            return carry

        lax.fori_loop(0, NCHUNK // 2, body, 0)
        for b in range(2):
            pltpu.make_async_copy(rows_v.at[b], emb_out.at[pl.ds(0, ECHUNK)],
                                  swe.at[b]).wait()
            pltpu.make_async_copy(linval_v.at[b],
                                  lin_out.at[pl.ds(0, LCHUNK)],
                                  swl.at[b]).wait()

    return k(xi_flat, xip_flat, emb_table, lin16)


DEEP_INP = FPAD * EMBED_DIM


def _tc_body(e0_ref, e1_ref, e2_ref, e3_ref, lin_ref, w1_ref, b1_ref, g_ref,
             bt_ref, w2_ref, b2_ref, out_ref, m_scr, v_scr, ss_scr):
    p = pl.program_id(0)
    i = pl.program_id(1)
    blk = jnp.concatenate(
        [e0_ref[...], e1_ref[...], e2_ref[...], e3_ref[...]],
        axis=1)
    h = jnp.dot(blk, w1_ref[...], preferred_element_type=jnp.float32,
                precision=lax.Precision.DEFAULT) + b1_ref[...]

    @pl.when(p == 0)
    def _phase0():
        m_k = jnp.mean(h, axis=0, keepdims=True)
        d = h - m_k
        m_scr[pl.ds(i, 1), :] = m_k
        v_scr[pl.ds(i, 1), :] = jnp.sum(d * d, axis=0, keepdims=True)

        @pl.when(i == NB - 1)
        def _finalize():
            mean = jnp.mean(m_scr[...], axis=0, keepdims=True)
            dm = m_scr[...] - mean
            var = (jnp.sum(v_scr[...], axis=0, keepdims=True)
                   + BB * jnp.sum(dm * dm, axis=0, keepdims=True)) / BATCH
            scale = g_ref[...] * lax.rsqrt(var + 1e-5)
            shift = bt_ref[...] - mean * scale
            ss_scr[0:1, :] = scale
            ss_scr[1:2, :] = shift

    @pl.when(p == 1)
    def _phase1():
        scale = ss_scr[0:1, :]
        shift = ss_scr[1:2, :]
        hn = jnp.maximum(h * scale + shift, 0.0)
        dblk = jnp.dot(hn, w2_ref[...], preferred_element_type=jnp.float32,
                       precision=lax.Precision.DEFAULT)
        f_ids = lax.broadcasted_iota(jnp.int32, (DEEP_INP, EMBED_DIM), 0)
        c_ids = lax.broadcasted_iota(jnp.int32, (DEEP_INP, EMBED_DIM), 1)
        sel = ((f_ids % EMBED_DIM == c_ids)
               & (f_ids < DEEP_IN)).astype(jnp.float32)
        s = jnp.dot(blk, sel, preferred_element_type=jnp.float32,
                    precision=lax.Precision.DEFAULT)
        fm = (lax.broadcasted_iota(jnp.int32, (1, DEEP_INP), 1)
              < DEEP_IN).astype(jnp.float32)
        ix = 0.5 * (jnp.sum(s * s, axis=1, keepdims=True)
                    - jnp.sum(blk * blk * fm, axis=1, keepdims=True))
        linv = jnp.sum(lin_ref[...], axis=1, keepdims=True)
        logit = dblk + b2_ref[...] + ix + linv
        out_ref[...] = 1.0 / (1.0 + jnp.exp(-logit))


def _tc_call(planes, lin2d, W1p, b1, gamma, beta, W2, b2):
    plane_spec = pl.BlockSpec((BB, 8 * EMBED_DIM), lambda p, i: (i, 0))
    return pl.pallas_call(
        _tc_body,
        grid=(2, NB),
        in_specs=[
            plane_spec, plane_spec, plane_spec, plane_spec,
            pl.BlockSpec((BB, NUM_FIELDS), lambda p, i: (i, 0)),
            pl.BlockSpec((DEEP_INP, DEEP_OUT), lambda p, i: (0, 0)),
            pl.BlockSpec((1, DEEP_OUT), lambda p, i: (0, 0)),
            pl.BlockSpec((1, DEEP_OUT), lambda p, i: (0, 0)),
            pl.BlockSpec((1, DEEP_OUT), lambda p, i: (0, 0)),
            pl.BlockSpec((DEEP_OUT, 1), lambda p, i: (0, 0)),
            pl.BlockSpec((1, 1), lambda p, i: (0, 0)),
        ],
        out_specs=pl.BlockSpec((BB, 1), lambda p, i: (i, 0)),
        out_shape=jax.ShapeDtypeStruct((BATCH, 1), jnp.float32),
        scratch_shapes=[
            pltpu.VMEM((NB, DEEP_OUT), jnp.float32),
            pltpu.VMEM((NB, DEEP_OUT), jnp.float32),
            pltpu.VMEM((8, DEEP_OUT), jnp.float32),
        ],
    )(*planes, lin2d, W1p, b1.reshape(1, -1), gamma.reshape(1, -1),
      beta.reshape(1, -1), W2, b2.reshape(1, 1))


def kernel(x, emb_table, lin_table, W1, b1, gamma, beta, W2, b2):
    offsets = (jnp.arange(NUM_FIELDS) * FIELD_DIM).astype(x.dtype)
    xi2d = (x + offsets[None, :]).astype(jnp.int32)
    xi = xi2d.reshape(-1)
    xi32 = jnp.pad(xi2d, ((0, 0), (0, FPAD - NUM_FIELDS)))
    xi_re = xi32.reshape(BATCH, 4, 8).transpose(1, 0, 2).reshape(-1)
    xi_p = _permute_idx(xi_re)
    lin16 = lin_table.reshape(-1, 16)
    emb128 = _tc_transpose(emb_table.T)
    emb_lin = emb128.reshape(-1, EMBED_DIM)
    emb_flat, lin_flat = _sc_gather(xi, xi_p, emb_lin, lin16)
    emb4 = emb_flat.reshape(4, BATCH, 8 * EMBED_DIM)
    planes = [emb4[q] for q in range(4)]
    lin2d = lin_flat.reshape(BATCH, NUM_FIELDS)
    W1p = jnp.pad(W1, ((0, DEEP_INP - DEEP_IN), (0, 0)))
    out2d = _tc_call(planes, lin2d, W1p, b1, gamma, beta, W2, b2)
    return out2d[:, 0]


# speedup vs baseline: 1.6607x; 1.6607x over previous
import functools

import jax
import jax.numpy as jnp
from jax import lax
from jax.experimental import pallas as pl
from jax.experimental.pallas import tpu as pltpu
from jax.experimental.pallas import tpu_sc as plsc

NUM_FIELDS = 26
FIELD_DIM = 100000
EMBED_DIM = 16
DEEP_IN = NUM_FIELDS * EMBED_DIM
DEEP_OUT = 400
BATCH = 16384
TOTAL_IDX = BATCH * NUM_FIELDS

NC = 2
NS = 16
NW = NC * NS
PER_W = TOTAL_IDX // NW
FPAD = 32
EMB_IDX = BATCH * FPAD
PER_WE = EMB_IDX // NW
ECHUNK = 512
LCHUNK = 416
NCHUNK = PER_WE // ECHUNK

BB = 1024
NB = BATCH // BB


TR_CB = 131072
TR_NB = (2600000 + TR_CB - 1) // TR_CB
TAB_ROWS_PAD = TR_NB * TR_CB


def _tr_body(in_ref, out_ref):
    x = in_ref[...]
    w8 = TR_CB // 8
    m = jnp.concatenate(
        [x[:, w8 * a:w8 * (a + 1)] for a in range(8)], axis=0)
    out_ref[...] = jnp.transpose(m, (1, 0))


def _tc_transpose(embT):
    return pl.pallas_call(
        _tr_body,
        grid=(TR_NB,),
        in_specs=[pl.BlockSpec((EMBED_DIM, TR_CB), lambda i: (0, i))],
        out_specs=pl.BlockSpec((TR_CB // 8, 128), lambda i: (i, 0)),
        out_shape=jax.ShapeDtypeStruct((TR_NB * (TR_CB // 8), 128), jnp.float32),
    )(embT)


def _permute_idx(r):
    return (r & ~(TR_CB - 1)) | ((r & (TR_CB // 8 - 1)) << 3) | ((r >> 14) & 7)


def _sc_gather(xi_flat, xip_flat, emb_table, lin16):
    mesh = plsc.VectorSubcoreMesh(core_axis_name="c", subcore_axis_name="s")

    @functools.partial(
        pl.kernel,
        mesh=mesh,
        compiler_params=pltpu.CompilerParams(use_tc_tiling_on_sc=False,
                                             needs_layout_passes=False),
        out_type=(
            jax.ShapeDtypeStruct((EMB_IDX, EMBED_DIM), jnp.float32),
            jax.ShapeDtypeStruct((TOTAL_IDX,), jnp.float32),
        ),
        scratch_types=[
            pltpu.VMEM((2, LCHUNK), jnp.int32),
            pltpu.VMEM((2, ECHUNK), jnp.int32),
            pltpu.VMEM((2, LCHUNK), jnp.int32),
            pltpu.VMEM((2, ECHUNK, EMBED_DIM), jnp.float32),
            pltpu.VMEM((2, LCHUNK, EMBED_DIM), jnp.float32),
            pltpu.VMEM((2, LCHUNK), jnp.float32),
            pltpu.SemaphoreType.DMA((2,)),
            pltpu.SemaphoreType.DMA((2,)),
            pltpu.SemaphoreType.DMA((2,)),
            pltpu.SemaphoreType.DMA((2,)),
            pltpu.SemaphoreType.DMA((2,)),
        ],
    )
    def k(xi_hbm, xip_hbm, emb_hbm, lin_hbm, emb_out, lin_out, idx_v, idxp_v,
          hi_v, rows_v, linrows_v, linval_v, sin, sge, sgl, swe, swl):
        wid = lax.axis_index("s") * NC + lax.axis_index("c")
        base_wl = wid * PER_W
        base_we = wid * PER_WE
        lanes = lax.iota(jnp.int32, 16)

        def fire_in(ci, b):
            pltpu.async_copy(xi_hbm.at[pl.ds(base_wl + ci * LCHUNK, LCHUNK)],
                             idx_v.at[b], sin.at[b])
            pltpu.async_copy(xip_hbm.at[pl.ds(base_we + ci * ECHUNK, ECHUNK)],
                             idxp_v.at[b], sin.at[b])

        fire_in(0, 0)

        def body(j, carry):
            for b in range(2):
                ci = 2 * j + b
                basel = base_wl + ci * LCHUNK
                basee = base_we + ci * ECHUNK
                pltpu.make_async_copy(xi_hbm.at[pl.ds(basel, LCHUNK)],
                                      idx_v.at[b], sin.at[b]).wait()
                pltpu.make_async_copy(xip_hbm.at[pl.ds(basee, ECHUNK)],
                                      idxp_v.at[b], sin.at[b]).wait()
                @pl.when(j > 0)
                def _():
                    pltpu.make_async_copy(rows_v.at[b],
                                          emb_out.at[pl.ds(0, ECHUNK)],
                                          swe.at[b]).wait()
                    pltpu.make_async_copy(linval_v.at[b],
                                          lin_out.at[pl.ds(0, LCHUNK)],
                                          swl.at[b]).wait()
                ge = pltpu.async_copy(emb_hbm.at[idxp_v.at[b]], rows_v.at[b],
                                      sge.at[b])

                def hi_body(g, c):
                    sl = pl.ds(g * 16, 16)
                    hi_v[b, sl] = lax.shift_right_logical(idx_v[b, sl], 4)
                    return c

                lax.fori_loop(0, LCHUNK // 16, hi_body, 0)
                gl = pltpu.async_copy(lin_hbm.at[hi_v.at[b]],
                                      linrows_v.at[b], sgl.at[b])
                @pl.when(ci + 1 < NCHUNK)
                def _():
                    fire_in(ci + 1, 1 - b)
                ge.wait()
                pltpu.async_copy(rows_v.at[b],
                                 emb_out.at[pl.ds(basee, ECHUNK)], swe.at[b])
                gl.wait()

                def sel_body(g, c):
                    sl = pl.ds(g * 16, 16)
                    lane = lax.bitwise_and(idx_v[b, sl], 15)
                    row = lanes + g * 16
                    linval_v[b, sl] = plsc.load_gather(linrows_v.at[b],
                                                       [row, lane])
                    return c

                lax.fori_loop(0, LCHUNK // 16, sel_body, 0)
                pltpu.async_copy(linval_v.at[b],
                                 lin_out.at[pl.ds(basel, LCHUNK)], swl.at[b])
            return carry

        lax.fori_loop(0, NCHUNK // 2, body, 0)
        for b in range(2):
            pltpu.make_async_copy(rows_v.at[b], emb_out.at[pl.ds(0, ECHUNK)],
                                  swe.at[b]).wait()
            pltpu.make_async_copy(linval_v.at[b],
                                  lin_out.at[pl.ds(0, LCHUNK)],
                                  swl.at[b]).wait()

    return k(xi_flat, xip_flat, emb_table, lin16)


DEEP_INP = FPAD * EMBED_DIM


def _tc_body(e0_ref, e1_ref, e2_ref, e3_ref, lin_ref, w1_ref, b1_ref, g_ref,
             bt_ref, w2_ref, b2_ref, out_ref, m_scr, v_scr, ss_scr):
    p = pl.program_id(0)
    i = pl.program_id(1)
    blk = jnp.concatenate(
        [e0_ref[...], e1_ref[...], e2_ref[...], e3_ref[...]],
        axis=1)
    h = jnp.dot(blk, w1_ref[...], preferred_element_type=jnp.float32,
                precision=lax.Precision.DEFAULT) + b1_ref[...]

    @pl.when(p == 0)
    def _phase0():
        m_k = jnp.mean(h, axis=0, keepdims=True)
        d = h - m_k
        m_scr[pl.ds(i, 1), :] = m_k
        v_scr[pl.ds(i, 1), :] = jnp.sum(d * d, axis=0, keepdims=True)

        @pl.when(i == NB - 1)
        def _finalize():
            mean = jnp.mean(m_scr[...], axis=0, keepdims=True)
            dm = m_scr[...] - mean
            var = (jnp.sum(v_scr[...], axis=0, keepdims=True)
                   + BB * jnp.sum(dm * dm, axis=0, keepdims=True)) / BATCH
            scale = g_ref[...] * lax.rsqrt(var + 1e-5)
            shift = bt_ref[...] - mean * scale
            ss_scr[0:1, :] = scale
            ss_scr[1:2, :] = shift

    @pl.when(p == 1)
    def _phase1():
        scale = ss_scr[0:1, :]
        shift = ss_scr[1:2, :]
        hn = jnp.maximum(h * scale + shift, 0.0)
        dblk = jnp.dot(hn, w2_ref[...], preferred_element_type=jnp.float32,
                       precision=lax.Precision.DEFAULT)
        f_ids = lax.broadcasted_iota(jnp.int32, (DEEP_INP, EMBED_DIM), 0)
        c_ids = lax.broadcasted_iota(jnp.int32, (DEEP_INP, EMBED_DIM), 1)
        sel = ((f_ids % EMBED_DIM == c_ids)
               & (f_ids < DEEP_IN)).astype(jnp.float32)
        s = jnp.dot(blk, sel, preferred_element_type=jnp.float32,
                    precision=lax.Precision.DEFAULT)
        fm = (lax.broadcasted_iota(jnp.int32, (1, DEEP_INP), 1)
              < DEEP_IN).astype(jnp.float32)
        ix = 0.5 * (jnp.sum(s * s, axis=1, keepdims=True)
                    - jnp.sum(blk * blk * fm, axis=1, keepdims=True))
        linv = jnp.sum(lin_ref[...], axis=1, keepdims=True)
        logit = dblk + b2_ref[...] + ix + linv
        out_ref[...] = 1.0 / (1.0 + jnp.exp(-logit))


def _tc_call(planes, lin2d, W1p, b1, gamma, beta, W2, b2):
    plane_spec = pl.BlockSpec((BB, 8 * EMBED_DIM), lambda p, i: (i, 0))
    return pl.pallas_call(
        _tc_body,
        grid=(2, NB),
        in_specs=[
            plane_spec, plane_spec, plane_spec, plane_spec,
            pl.BlockSpec((BB, NUM_FIELDS), lambda p, i: (i, 0)),
            pl.BlockSpec((DEEP_INP, DEEP_OUT), lambda p, i: (0, 0)),
            pl.BlockSpec((1, DEEP_OUT), lambda p, i: (0, 0)),
            pl.BlockSpec((1, DEEP_OUT), lambda p, i: (0, 0)),
            pl.BlockSpec((1, DEEP_OUT), lambda p, i: (0, 0)),
            pl.BlockSpec((DEEP_OUT, 1), lambda p, i: (0, 0)),
            pl.BlockSpec((1, 1), lambda p, i: (0, 0)),
        ],
        out_specs=pl.BlockSpec((BB, 1), lambda p, i: (i, 0)),
        out_shape=jax.ShapeDtypeStruct((BATCH, 1), jnp.float32),
        scratch_shapes=[
            pltpu.VMEM((NB, DEEP_OUT), jnp.float32),
            pltpu.VMEM((NB, DEEP_OUT), jnp.float32),
            pltpu.VMEM((8, DEEP_OUT), jnp.float32),
        ],
    )(*planes, lin2d, W1p, b1.reshape(1, -1), gamma.reshape(1, -1),
      beta.reshape(1, -1), W2, b2.reshape(1, 1))


def kernel(x, emb_table, lin_table, W1, b1, gamma, beta, W2, b2):
    offsets = (jnp.arange(NUM_FIELDS) * FIELD_DIM).astype(x.dtype)
    xi2d = (x + offsets[None, :]).astype(jnp.int32)
    xi = xi2d.reshape(-1)
    xi32 = jnp.concatenate([xi2d, xi2d[:, :FPAD - NUM_FIELDS]], axis=1)
    xi_re = xi32.reshape(BATCH, 4, 8).transpose(1, 0, 2).reshape(-1)
    xi_p = _permute_idx(xi_re)
    lin16 = lin_table.reshape(-1, 16)
    emb128 = _tc_transpose(emb_table.T)
    emb_lin = emb128.reshape(-1, EMBED_DIM)
    emb_flat, lin_flat = _sc_gather(xi, xi_p, emb_lin, lin16)
    emb4 = emb_flat.reshape(4, BATCH, 8 * EMBED_DIM)
    planes = [emb4[q] for q in range(4)]
    lin2d = lin_flat.reshape(BATCH, NUM_FIELDS)
    W1p = jnp.pad(W1, ((0, DEEP_INP - DEEP_IN), (0, 0)))
    out2d = _tc_call(planes, lin2d, W1p, b1, gamma, beta, W2, b2)
    return out2d[:, 0]


# speedup vs baseline: 2.7333x; 1.6459x over previous
import functools

import jax
import jax.numpy as jnp
from jax import lax
from jax.experimental import pallas as pl
from jax.experimental.pallas import tpu as pltpu
from jax.experimental.pallas import tpu_sc as plsc

NUM_FIELDS = 26
FIELD_DIM = 100000
EMBED_DIM = 16
DEEP_IN = NUM_FIELDS * EMBED_DIM
DEEP_OUT = 400
BATCH = 16384
TOTAL_IDX = BATCH * NUM_FIELDS

NC = 2
NS = 16
NW = NC * NS
PER_W = TOTAL_IDX // NW
CHUNK = 512
NCHUNK = PER_W // CHUNK

BB = 1024
NB = BATCH // BB


TR_CB = 131072
TR_NB = (2600000 + TR_CB - 1) // TR_CB
TAB_ROWS_PAD = TR_NB * TR_CB


def _tr_body(in_ref, out_ref):
    x = in_ref[...]
    w8 = TR_CB // 8
    m = jnp.concatenate(
        [x[:, w8 * a:w8 * (a + 1)] for a in range(8)], axis=0)
    out_ref[...] = jnp.transpose(m, (1, 0))


def _tc_transpose(embT):
    return pl.pallas_call(
        _tr_body,
        grid=(TR_NB,),
        in_specs=[pl.BlockSpec((EMBED_DIM, TR_CB), lambda i: (0, i))],
        out_specs=pl.BlockSpec((TR_CB // 8, 128), lambda i: (i, 0)),
        out_shape=jax.ShapeDtypeStruct((TR_NB * (TR_CB // 8), 128), jnp.float32),
    )(embT)


def _permute_idx(r):
    return (r & ~(TR_CB - 1)) | ((r & (TR_CB // 8 - 1)) << 3) | ((r >> 14) & 7)


def _sc_gather(xi_flat, xip_flat, emb_table, lin16):
    mesh = plsc.VectorSubcoreMesh(core_axis_name="c", subcore_axis_name="s")

    @functools.partial(
        pl.kernel,
        mesh=mesh,
        compiler_params=pltpu.CompilerParams(use_tc_tiling_on_sc=False,
                                             needs_layout_passes=False),
        out_type=(
            jax.ShapeDtypeStruct((TOTAL_IDX, EMBED_DIM), jnp.float32),
            jax.ShapeDtypeStruct((TOTAL_IDX,), jnp.float32),
        ),
        scratch_types=[
            pltpu.VMEM((2, CHUNK), jnp.int32),
            pltpu.VMEM((2, CHUNK), jnp.int32),
            pltpu.VMEM((2, CHUNK), jnp.int32),
            pltpu.VMEM((2, CHUNK, EMBED_DIM), jnp.float32),
            pltpu.VMEM((2, CHUNK, EMBED_DIM), jnp.float32),
            pltpu.VMEM((2, CHUNK), jnp.float32),
            pltpu.SemaphoreType.DMA((2,)),
            pltpu.SemaphoreType.DMA((2,)),
            pltpu.SemaphoreType.DMA((2,)),
            pltpu.SemaphoreType.DMA((2,)),
            pltpu.SemaphoreType.DMA((2,)),
        ],
    )
    def k(xi_hbm, xip_hbm, emb_hbm, lin_hbm, emb_out, lin_out, idx_v, idxp_v,
          hi_v, rows_v, linrows_v, linval_v, sin, sge, sgl, swe, swl):
        wid = lax.axis_index("s") * NC + lax.axis_index("c")
        base_w = wid * PER_W
        lanes = lax.iota(jnp.int32, 16)

        def fire_in(ci, b):
            base = base_w + ci * CHUNK
            c1 = pltpu.async_copy(xi_hbm.at[pl.ds(base, CHUNK)], idx_v.at[b],
                                  sin.at[b])
            c2 = pltpu.async_copy(xip_hbm.at[pl.ds(base, CHUNK)],
                                  idxp_v.at[b], sin.at[b])
            return c1, c2

        p1, p2 = fire_in(0, 0)

        def body(j, carry):
            for b in range(2):
                ci = 2 * j + b
                base = base_w + ci * CHUNK
                pltpu.make_async_copy(xi_hbm.at[pl.ds(base, CHUNK)],
                                      idx_v.at[b], sin.at[b]).wait()
                pltpu.make_async_copy(xip_hbm.at[pl.ds(base, CHUNK)],
                                      idxp_v.at[b], sin.at[b]).wait()
                @pl.when(j > 0)
                def _():
                    pltpu.make_async_copy(rows_v.at[b],
                                          emb_out.at[pl.ds(0, CHUNK)],
                                          swe.at[b]).wait()
                    pltpu.make_async_copy(linval_v.at[b],
                                          lin_out.at[pl.ds(0, CHUNK)],
                                          swl.at[b]).wait()
                ge = pltpu.async_copy(emb_hbm.at[idxp_v.at[b]], rows_v.at[b],
                                      sge.at[b])

                def hi_body(g, c):
                    sl = pl.ds(g * 16, 16)
                    hi_v[b, sl] = lax.shift_right_logical(idx_v[b, sl], 4)
                    return c

                lax.fori_loop(0, CHUNK // 16, hi_body, 0)
                gl = pltpu.async_copy(lin_hbm.at[hi_v.at[b]],
                                      linrows_v.at[b], sgl.at[b])
                @pl.when(ci + 1 < NCHUNK)
                def _():
                    nb = base_w + (ci + 1) * CHUNK
                    pltpu.async_copy(xi_hbm.at[pl.ds(nb, CHUNK)],
                                     idx_v.at[1 - b], sin.at[1 - b])
                    pltpu.async_copy(xip_hbm.at[pl.ds(nb, CHUNK)],
                                     idxp_v.at[1 - b], sin.at[1 - b])
                ge.wait()
                pltpu.async_copy(rows_v.at[b], emb_out.at[pl.ds(base, CHUNK)],
                                 swe.at[b])
                gl.wait()

                def sel_body(g, c):
                    sl = pl.ds(g * 16, 16)
                    lane = lax.bitwise_and(idx_v[b, sl], 15)
                    row = lanes + g * 16
                    linval_v[b, sl] = plsc.load_gather(linrows_v.at[b],
                                                       [row, lane])
                    return c

                lax.fori_loop(0, CHUNK // 16, sel_body, 0)
                pltpu.async_copy(linval_v.at[b],
                                 lin_out.at[pl.ds(base, CHUNK)], swl.at[b])
            return carry

        lax.fori_loop(0, NCHUNK // 2, body, 0)
        for b in range(2):
            pltpu.make_async_copy(rows_v.at[b], emb_out.at[pl.ds(0, CHUNK)],
                                  swe.at[b]).wait()
            pltpu.make_async_copy(linval_v.at[b],
                                  lin_out.at[pl.ds(0, CHUNK)],
                                  swl.at[b]).wait()

    return k(xi_flat, xip_flat, emb_table, lin16)


def _tc_body(emb_ref, lin_ref, w1_ref, b1_ref, g_ref, bt_ref, w2_ref, b2_ref,
             out_ref, m_scr, v_scr, ss_scr):
    p = pl.program_id(0)
    i = pl.program_id(1)
    blk = emb_ref[...]
    h = jnp.dot(blk, w1_ref[...], preferred_element_type=jnp.float32,
                precision=lax.Precision.DEFAULT) + b1_ref[...]

    @pl.when(p == 0)
    def _phase0():
        m_k = jnp.mean(h, axis=0, keepdims=True)
        d = h - m_k
        m_scr[pl.ds(i, 1), :] = m_k
        v_scr[pl.ds(i, 1), :] = jnp.sum(d * d, axis=0, keepdims=True)

        @pl.when(i == NB - 1)
        def _finalize():
            mean = jnp.mean(m_scr[...], axis=0, keepdims=True)
            dm = m_scr[...] - mean
            var = (jnp.sum(v_scr[...], axis=0, keepdims=True)
                   + BB * jnp.sum(dm * dm, axis=0, keepdims=True)) / BATCH
            scale = g_ref[...] * lax.rsqrt(var + 1e-5)
            shift = bt_ref[...] - mean * scale
            ss_scr[0:1, :] = scale
            ss_scr[1:2, :] = shift

    @pl.when(p == 1)
    def _phase1():
        scale = ss_scr[0:1, :]
        shift = ss_scr[1:2, :]
        hn = jnp.maximum(h * scale + shift, 0.0)
        dblk = jnp.dot(hn, w2_ref[...], preferred_element_type=jnp.float32,
                       precision=lax.Precision.DEFAULT)
        f_ids = lax.broadcasted_iota(jnp.int32, (DEEP_IN, EMBED_DIM), 0)
        c_ids = lax.broadcasted_iota(jnp.int32, (DEEP_IN, EMBED_DIM), 1)
        sel = (f_ids % EMBED_DIM == c_ids).astype(jnp.float32)
        s = jnp.dot(blk, sel, preferred_element_type=jnp.float32,
                    precision=lax.Precision.DEFAULT)
        ix = 0.5 * (jnp.sum(s * s, axis=1, keepdims=True)
                    - jnp.sum(blk * blk, axis=1, keepdims=True))
        linv = jnp.sum(lin_ref[...], axis=1, keepdims=True)
        logit = dblk + b2_ref[...] + ix + linv
        out_ref[...] = 1.0 / (1.0 + jnp.exp(-logit))


def _tc_call(emb2d, lin2d, W1, b1, gamma, beta, W2, b2):
    return pl.pallas_call(
        _tc_body,
        grid=(2, NB),
        in_specs=[
            pl.BlockSpec((BB, DEEP_IN), lambda p, i: (i, 0)),
            pl.BlockSpec((BB, NUM_FIELDS), lambda p, i: (i, 0)),
            pl.BlockSpec((DEEP_IN, DEEP_OUT), lambda p, i: (0, 0)),
            pl.BlockSpec((1, DEEP_OUT), lambda p, i: (0, 0)),
            pl.BlockSpec((1, DEEP_OUT), lambda p, i: (0, 0)),
            pl.BlockSpec((1, DEEP_OUT), lambda p, i: (0, 0)),
            pl.BlockSpec((DEEP_OUT, 1), lambda p, i: (0, 0)),
            pl.BlockSpec((1, 1), lambda p, i: (0, 0)),
        ],
        out_specs=pl.BlockSpec((BB, 1), lambda p, i: (i, 0)),
        out_shape=jax.ShapeDtypeStruct((BATCH, 1), jnp.float32),
        scratch_shapes=[
            pltpu.VMEM((NB, DEEP_OUT), jnp.float32),
            pltpu.VMEM((NB, DEEP_OUT), jnp.float32),
            pltpu.VMEM((8, DEEP_OUT), jnp.float32),
        ],
    )(emb2d, lin2d, W1, b1.reshape(1, -1), gamma.reshape(1, -1),
      beta.reshape(1, -1), W2, b2.reshape(1, 1))


def kernel(x, emb_table, lin_table, W1, b1, gamma, beta, W2, b2):
    offsets = (jnp.arange(NUM_FIELDS) * FIELD_DIM).astype(x.dtype)
    xi = (x + offsets[None, :]).astype(jnp.int32).reshape(-1)
    lin16 = lin_table.reshape(-1, 16)
    emb128 = _tc_transpose(emb_table.T)
    emb_lin = emb128.reshape(-1, EMBED_DIM)
    xi_p = _permute_idx(xi)
    emb_flat, lin_flat = _sc_gather(xi, xi_p, emb_lin, lin16)
    emb2d = emb_flat.reshape(BATCH, DEEP_IN)
    lin2d = lin_flat.reshape(BATCH, NUM_FIELDS)
    out2d = _tc_call(emb2d, lin2d, W1, b1, gamma, beta, W2, b2)
    return out2d[:, 0]
